# Initial kernel scaffold; baseline (speedup 1.0000x reference)
import jax
import jax.numpy as jnp
from jax.experimental import pallas as pl


def kernel(t, y):
    raise NotImplementedError("write your pallas kernel here")


# speedup vs baseline: 1.9882x; 1.9882x over previous
import jax
import jax.numpy as jnp
from jax.experimental import pallas as pl

_L = 10.0
_V0 = 0.05
_BETA = 1.0
_A_CF = 1.0
_A_CIL = 1.0
_R = 1.0
_A = 0.1
_D_MAC = 1.0
_N = 2048
_BM = 256


def _drift_block(y_ref, yt_ref, o_ref):
    xi = y_ref[:, 0:1]
    yi = y_ref[:, 1:2]
    thi = y_ref[:, 2:3]
    xj = yt_ref[0:1, :]
    yj = yt_ref[1:2, :]
    thj = yt_ref[2:3, :]

    cj = jnp.cos(thj)
    sj = jnp.sin(thj)

    dx = xi - xj
    dx = dx - _L * jnp.round(dx * (1.0 / _L))
    dy = yi - yj
    dy = dy - _L * jnp.round(dy * (1.0 / _L))

    d2 = dx * dx + dy * dy
    d = jnp.sqrt(d2 + 1e-12)
    mask = (d < _R).astype(jnp.float32)
    inv = mask / d
    wcil = inv * (1.0 - d * (1.0 / _R))
    wchem = inv * jnp.exp(-d)

    jcil_x = jnp.sum(dx * wcil, axis=1, keepdims=True)
    jcil_y = jnp.sum(dy * wcil, axis=1, keepdims=True)
    jchem_x = jnp.sum(dx * wchem, axis=1, keepdims=True)
    jchem_y = jnp.sum(dy * wchem, axis=1, keepdims=True)
    jcf_x = jnp.sum(mask * cj, axis=1, keepdims=True)
    jcf_y = jnp.sum(mask * sj, axis=1, keepdims=True)

    ci = jnp.cos(thi)
    si = jnp.sin(thi)
    vx = _A_CF * jcf_x - _A_CIL * jcil_x
    vy = _A_CF * jcf_y - _A_CIL * jcil_y
    dth = ci * vy - si * vx + _A * ci + _D_MAC * (ci * jchem_y - si * jchem_x)
    ox = _V0 * ci - _BETA * jcil_x
    oy = _V0 * si - _BETA * jcil_y
    o_ref[:, :] = jnp.concatenate([ox, oy, dth], axis=1)


@jax.jit
def _drift(y):
    yt = y.T
    return pl.pallas_call(
        _drift_block,
        grid=(_N // _BM,),
        in_specs=[
            pl.BlockSpec((_BM, 3), lambda i: (i, 0)),
            pl.BlockSpec((3, _N), lambda i: (0, 0)),
        ],
        out_specs=pl.BlockSpec((_BM, 3), lambda i: (i, 0)),
        out_shape=jax.ShapeDtypeStruct((_N, 3), jnp.float32),
    )(y, yt)


def kernel(t, y):
    return _drift(y)


# speedup vs baseline: 2.0523x; 1.0322x over previous
import jax
import jax.numpy as jnp
from jax.experimental import pallas as pl
from jax.experimental.pallas import tpu as pltpu

_L = 10.0
_V0 = 0.05
_BETA = 1.0
_A_CF = 1.0
_A_CIL = 1.0
_R = 1.0
_A = 0.1
_D_MAC = 1.0
_N = 2048
_BM = 256


def _drift_block(y_ref, yt_ref, o_ref):
    xi = y_ref[:, 0:1]
    yi = y_ref[:, 1:2]
    thi = y_ref[:, 2:3]
    xj = yt_ref[0:1, :]
    yj = yt_ref[1:2, :]
    thj = yt_ref[2:3, :]

    ci = jnp.cos(thi)
    si = jnp.sin(thi)
    cj = jnp.cos(thj)
    sj = jnp.sin(thj)

    dx = xi - xj
    dx = dx - _L * jnp.round(dx * (1.0 / _L))
    dy = yi - yj
    dy = dy - _L * jnp.round(dy * (1.0 / _L))

    r2 = dx * dx + dy * dy + 1e-12
    mask = (r2 < _R * _R).astype(jnp.float32)
    inv = mask * jax.lax.rsqrt(r2)
    d = r2 * inv
    wcil = inv * (1.0 - d)
    wdiff = inv * jnp.exp(-d) - wcil

    tx = mask * cj + dx * wdiff
    ty = mask * sj + dy * wdiff
    ang = ci * ty - si * tx

    jcil_x = jnp.sum(dx * wcil, axis=1, keepdims=True)
    jcil_y = jnp.sum(dy * wcil, axis=1, keepdims=True)
    jang = jnp.sum(ang, axis=1, keepdims=True)

    dth = jang + _A * ci
    ox = _V0 * ci - _BETA * jcil_x
    oy = _V0 * si - _BETA * jcil_y
    o_ref[:, :] = jnp.concatenate([ox, oy, dth], axis=1)


@jax.jit
def _drift(y):
    yt = y.T
    return pl.pallas_call(
        _drift_block,
        grid=(_N // _BM,),
        in_specs=[
            pl.BlockSpec((_BM, 3), lambda i: (i, 0)),
            pl.BlockSpec((3, _N), lambda i: (0, 0)),
        ],
        out_specs=pl.BlockSpec((_BM, 3), lambda i: (i, 0)),
        out_shape=jax.ShapeDtypeStruct((_N, 3), jnp.float32),
        compiler_params=pltpu.CompilerParams(
            dimension_semantics=("parallel",),
        ),
    )(y, yt)


def kernel(t, y):
    return _drift(y)
